# Initial kernel scaffold; baseline (speedup 1.0000x reference)
import jax
import jax.numpy as jnp
from jax.experimental import pallas as pl


def kernel(x, pos, pre_table, w1, b1, g1, be1, g2, be2, W, bL):
    raise NotImplementedError("write your pallas kernel here")


# speedup vs baseline: 1.0020x; 1.0020x over previous
import functools

import jax
import jax.numpy as jnp
from jax.experimental import pallas as pl
from jax.experimental.pallas import tpu as pltpu

REL_POS_WIDTH = 16
TABLE_WIDTH = 2 * REL_POS_WIDTH + 1
K = 9
INNER = 4


def _ln_matmul_kernel(f_ref, g2_ref, be2_ref, w_ref, bl_ref, o_ref):
    f = f_ref[...]
    m = jnp.mean(f, axis=-1, keepdims=True)
    d = f - m
    v = jnp.mean(d * d, axis=-1, keepdims=True)
    y = d * jax.lax.rsqrt(v + 1e-5) * g2_ref[...] + be2_ref[...]
    o_ref[...] = (
        jnp.dot(y, w_ref[...], preferred_element_type=jnp.float32) + bl_ref[...]
    )


def _ln_matmul(feat, g2, be2, W, bL):
    M, C_in = feat.shape
    C_out = W.shape[1]
    R = 800
    grid = (M // R,)
    return pl.pallas_call(
        _ln_matmul_kernel,
        grid=grid,
        in_specs=[
            pl.BlockSpec((R, C_in), lambda i: (i, 0)),
            pl.BlockSpec((C_in,), lambda i: (0,)),
            pl.BlockSpec((C_in,), lambda i: (0,)),
            pl.BlockSpec((C_in, C_out), lambda i: (0, 0)),
            pl.BlockSpec((C_out,), lambda i: (0,)),
        ],
        out_specs=pl.BlockSpec((R, C_out), lambda i: (i, 0)),
        out_shape=jax.ShapeDtypeStruct((M, C_out), jnp.float32),
    )(feat, g2, be2, W, bL)


def _knn(pos, k):
    b, n, _ = pos.shape
    chunk = 1000

    def per_batch(p):
        def per_chunk(q):
            d = jnp.sum((q[:, None, :] - p[None, :, :]) ** 2, axis=-1)
            _, idx = jax.lax.top_k(-d, k)
            return idx

        qs = p.reshape(n // chunk, chunk, 2)
        return jax.lax.map(per_chunk, qs).reshape(n, k)

    return jax.vmap(per_batch)(pos)


def _layer_norm(x, g, b, eps=1e-5):
    m = jnp.mean(x, axis=-1, keepdims=True)
    v = jnp.var(x, axis=-1, keepdims=True)
    return (x - m) / jnp.sqrt(v + eps) * g + b


def kernel(x, pos, pre_table, w1, b1, g1, be1, g2, be2, W, bL):
    b, n, c = x.shape
    nn_idx = _knn(pos, K)
    nn_pos = jax.vmap(lambda p, i: p[i])(pos, nn_idx)
    rel_pos = pos[:, :, None, :] - nn_pos
    weights_table = jax.nn.gelu(
        _layer_norm(pre_table @ w1 + b1, g1, be1), approximate=False
    )
    relq = jnp.clip(rel_pos.astype(jnp.int32) + REL_POS_WIDTH, 0, TABLE_WIDTH - 1)
    pe_idx = relq[..., 1] * TABLE_WIDTH + relq[..., 0]
    weights = weights_table[pe_idx]
    xg = jax.vmap(lambda f, i: f[i])(x, nn_idx)
    feat = jnp.einsum("bnki,bnkc->bnic", weights, xg).reshape(b * n, INNER * c)
    out = _ln_matmul(feat, g2, be2, W, bL)
    return out.reshape(b, n, W.shape[1])


# speedup vs baseline: 17.0808x; 17.0476x over previous
import functools

import jax
import jax.numpy as jnp
from jax.experimental import pallas as pl
from jax.experimental.pallas import tpu as pltpu

REL_POS_WIDTH = 16
TABLE_WIDTH = 2 * REL_POS_WIDTH + 1
K = 9
INNER = 4


def _ln_matmul_kernel(f_ref, g2_ref, be2_ref, w_ref, bl_ref, o_ref):
    f = f_ref[...]
    m = jnp.mean(f, axis=-1, keepdims=True)
    d = f - m
    v = jnp.mean(d * d, axis=-1, keepdims=True)
    y = d * jax.lax.rsqrt(v + 1e-5) * g2_ref[...] + be2_ref[...]
    o_ref[...] = (
        jnp.dot(y, w_ref[...], preferred_element_type=jnp.float32) + bl_ref[...]
    )


def _ln_matmul(feat, g2, be2, W, bL):
    M, C_in = feat.shape
    C_out = W.shape[1]
    R = next(r for r in (800, 512, 256, 128, 64, 32, 16, 8) if M % r == 0)
    grid = (M // R,)
    return pl.pallas_call(
        _ln_matmul_kernel,
        grid=grid,
        in_specs=[
            pl.BlockSpec((R, C_in), lambda i: (i, 0)),
            pl.BlockSpec((C_in,), lambda i: (0,)),
            pl.BlockSpec((C_in,), lambda i: (0,)),
            pl.BlockSpec((C_in, C_out), lambda i: (0, 0)),
            pl.BlockSpec((C_out,), lambda i: (0,)),
        ],
        out_specs=pl.BlockSpec((R, C_out), lambda i: (i, 0)),
        out_shape=jax.ShapeDtypeStruct((M, C_out), jnp.float32),
    )(feat, g2, be2, W, bL)


_QT = 64
_CW = 128
_CAP = 4
_KSEL = 12
_IMAX = 0x7FFFFFFF


def _knn_kernel(qpos_ref, cand_ref, o_ref):
    qp = qpos_ref[0]
    qx = qp[:, 0:1]
    qy = qp[:, 1:2]
    nchunks = cand_ref.shape[2]

    def body(j, accs):
        px = cand_ref[0, 0:1, j, :]
        py = cand_ref[0, 1:2, j, :]
        dx = qx - px
        dy = qy - py
        d = dx * dx + dy * dy
        v = (jax.lax.bitcast_convert_type(d, jnp.int32) & jnp.int32(-128)) | j
        out = []
        for a in accs:
            lo = jnp.minimum(a, v)
            v = jnp.maximum(a, v)
            out.append(lo)
        return tuple(out)

    init = tuple(
        jnp.full((qp.shape[0], _CW), _IMAX, jnp.int32) for _ in range(_CAP)
    )
    accs = jax.lax.fori_loop(0, nchunks, body, init)

    m = jnp.concatenate(accs, axis=1)
    col_iota = jax.lax.broadcasted_iota(jnp.int32, m.shape, 1)
    cols = []
    for _ in range(_KSEL):
        best = jnp.min(m, axis=1, keepdims=True)
        sel = jnp.where(m == best, col_iota, _IMAX)
        col = jnp.min(sel, axis=1, keepdims=True)
        cols.append((best & 127) * _CW + (col & (_CW - 1)))
        m = jnp.where(col_iota == col, _IMAX, m)
    pad = jnp.zeros((qp.shape[0], 16 - _KSEL), jnp.int32)
    o_ref[0] = jnp.concatenate(cols + [pad], axis=1)


def _knn(pos, k):
    b, n, _ = pos.shape
    npad = ((n + _CW - 1) // _CW) * _CW
    nchunks = npad // _CW
    pos_pad = jnp.pad(pos, ((0, 0), (0, npad - n), (0, 0)),
                      constant_values=1e9)
    cand = pos_pad.transpose(0, 2, 1).reshape(b, 2, nchunks, _CW)
    grid = (b, npad // _QT)
    out = pl.pallas_call(
        _knn_kernel,
        grid=grid,
        in_specs=[
            pl.BlockSpec((1, _QT, 2), lambda i, j: (i, j, 0)),
            pl.BlockSpec((1, 2, nchunks, _CW), lambda i, j: (i, 0, 0, 0)),
        ],
        out_specs=pl.BlockSpec((1, _QT, 16), lambda i, j: (i, j, 0)),
        out_shape=jax.ShapeDtypeStruct((b, npad, 16), jnp.int32),
    )(pos_pad, cand)
    idx12 = out[:, :n, :_KSEL]
    cand_pos = jax.vmap(lambda p, i: p[i])(pos, idx12)
    dd = jnp.sum((pos[:, :, None, :] - cand_pos) ** 2, axis=-1)
    _, sel = jax.lax.top_k(-dd, k)
    return jnp.take_along_axis(idx12, sel, axis=-1)


def _layer_norm(x, g, b, eps=1e-5):
    m = jnp.mean(x, axis=-1, keepdims=True)
    v = jnp.var(x, axis=-1, keepdims=True)
    return (x - m) / jnp.sqrt(v + eps) * g + b


def kernel(x, pos, pre_table, w1, b1, g1, be1, g2, be2, W, bL):
    b, n, c = x.shape
    nn_idx = _knn(pos, K)
    nn_pos = jax.vmap(lambda p, i: p[i])(pos, nn_idx)
    rel_pos = pos[:, :, None, :] - nn_pos
    weights_table = jax.nn.gelu(
        _layer_norm(pre_table @ w1 + b1, g1, be1), approximate=False
    )
    relq = jnp.clip(rel_pos.astype(jnp.int32) + REL_POS_WIDTH, 0, TABLE_WIDTH - 1)
    pe_idx = relq[..., 1] * TABLE_WIDTH + relq[..., 0]
    weights = weights_table[pe_idx]
    xg = jax.vmap(lambda f, i: f[i])(x, nn_idx)
    feat = jnp.einsum("bnki,bnkc->bnic", weights, xg).reshape(b * n, INNER * c)
    out = _ln_matmul(feat, g2, be2, W, bL)
    return out.reshape(b, n, W.shape[1])


# speedup vs baseline: 41.9989x; 2.4588x over previous
import functools

import jax
import jax.numpy as jnp
from jax import lax
from jax.experimental import pallas as pl
from jax.experimental.pallas import tpu as pltpu
from jax.experimental.pallas import tpu_sc as plsc

REL_POS_WIDTH = 16
TABLE_WIDTH = 2 * REL_POS_WIDTH + 1
K = 9
INNER = 4

_QT = 64
_CW = 128
_CAP = 4
_KSEL = 12
_IMAX = 0x7FFFFFFF

_NW = 32
_SG = 10


def _knn_kernel(qpos_ref, cand_ref, o_ref):
    qp = qpos_ref[0]
    qx = qp[:, 0:1]
    qy = qp[:, 1:2]
    nchunks = cand_ref.shape[2]

    def body(j, accs):
        px = cand_ref[0, 0:1, j, :]
        py = cand_ref[0, 1:2, j, :]
        dx = qx - px
        dy = qy - py
        d = dx * dx + dy * dy
        v = (lax.bitcast_convert_type(d, jnp.int32) & jnp.int32(-128)) | j
        out = []
        for a in accs:
            lo = jnp.minimum(a, v)
            v = jnp.maximum(a, v)
            out.append(lo)
        return tuple(out)

    init = tuple(
        jnp.full((qp.shape[0], _CW), _IMAX, jnp.int32) for _ in range(_CAP)
    )
    accs = lax.fori_loop(0, nchunks, body, init)

    m = jnp.concatenate(accs, axis=1)
    col_iota = lax.broadcasted_iota(jnp.int32, m.shape, 1)
    cols = []
    for _ in range(_KSEL):
        best = jnp.min(m, axis=1, keepdims=True)
        sel = jnp.where(m == best, col_iota, _IMAX)
        col = jnp.min(sel, axis=1, keepdims=True)
        cols.append((best & 127) * _CW + (col & (_CW - 1)))
        m = jnp.where(col_iota == col, _IMAX, m)
    pad = jnp.zeros((qp.shape[0], 16 - _KSEL), jnp.int32)
    o_ref[0] = jnp.concatenate(cols + [pad], axis=1)


def _knn12(pos):
    b, n, _ = pos.shape
    npad = ((n + _CW - 1) // _CW) * _CW
    nchunks = npad // _CW
    pos_pad = jnp.pad(pos, ((0, 0), (0, npad - n), (0, 0)),
                      constant_values=1e9)
    cand = pos_pad.transpose(0, 2, 1).reshape(b, 2, nchunks, _CW)
    grid = (b, npad // _QT)
    out = pl.pallas_call(
        _knn_kernel,
        grid=grid,
        in_specs=[
            pl.BlockSpec((1, _QT, 2), lambda i, j: (i, j, 0)),
            pl.BlockSpec((1, 2, nchunks, _CW), lambda i, j: (i, 0, 0, 0)),
        ],
        out_specs=pl.BlockSpec((1, _QT, 16), lambda i, j: (i, j, 0)),
        out_shape=jax.ShapeDtypeStruct((b, npad, 16), jnp.int32),
    )(pos_pad, cand)
    return out[:, :n, :]


def _wt_kernel(pt_ref, w1_ref, b1_ref, g1_ref, be1_ref, o_ref):
    t = [None] * INNER
    for j in range(INNER):
        acc = pt_ref[0:1, :] * w1_ref[0, j]
        for mm in range(1, 5):
            acc = acc + pt_ref[mm:mm + 1, :] * w1_ref[mm, j]
        t[j] = acc + b1_ref[0, j]
    mu = (t[0] + t[1] + t[2] + t[3]) * 0.25
    dev = [tj - mu for tj in t]
    var = (dev[0] * dev[0] + dev[1] * dev[1]
           + dev[2] * dev[2] + dev[3] * dev[3]) * 0.25
    rs = lax.rsqrt(var + 1e-5)
    for j in range(INNER):
        y = dev[j] * rs * g1_ref[0, j] + be1_ref[0, j]
        o_ref[j:j + 1, :] = y * 0.5 * (1.0 + lax.erf(y * (2.0 ** -0.5)))


_T2PAD = 1152


def _weights_table(pre_table, w1, b1, g1, be1):
    t2 = pre_table.shape[0]
    pt = jnp.pad(pre_table.T, ((0, 3), (0, _T2PAD - t2)))
    w1p = jnp.zeros((8, 8), jnp.float32).at[:5, :INNER].set(w1)
    return pl.pallas_call(
        _wt_kernel,
        in_specs=[
            pl.BlockSpec((8, _T2PAD), lambda: (0, 0)),
            pl.BlockSpec((8, 8), lambda: (0, 0)),
            pl.BlockSpec((1, 8), lambda: (0, 0)),
            pl.BlockSpec((1, 8), lambda: (0, 0)),
            pl.BlockSpec((1, 8), lambda: (0, 0)),
        ],
        out_specs=pl.BlockSpec((8, _T2PAD), lambda: (0, 0)),
        out_shape=jax.ShapeDtypeStruct((8, _T2PAD), jnp.float32),
    )(pt, w1p, jnp.pad(b1, (0, 4)).reshape(1, 8),
      jnp.pad(g1, (0, 4)).reshape(1, 8), jnp.pad(be1, (0, 4)).reshape(1, 8))


def _sc_feat_body(x_hbm, posx_hbm, posy_hbm, idx_hbm, wt_hbm, out_hbm,
                  posx_v, posy_v, ibuf, idxc, wt_v, wbuf, ring, fbuf,
                  sem_in0, sem_in1, sem_out0, sem_out1):
    n_total, dim = x_hbm.shape
    span = n_total // _NW
    n_per_b = posx_hbm.shape[1]
    ng = span // _SG
    ichunk = ibuf.shape[0]

    wid = lax.axis_index("s") * 2 + lax.axis_index("c")
    r0 = wid * span
    bb = r0 // n_per_b
    b_base = bb * n_per_b

    pltpu.sync_copy(posx_hbm.at[bb], posx_v)
    pltpu.sync_copy(posy_hbm.at[bb], posy_v)
    pltpu.sync_copy(wt_hbm, wt_v)

    lane = lax.iota(jnp.int32, 16)
    inf = jnp.float32(jnp.inf)
    zeros16 = jnp.zeros((16,), jnp.int32)

    def padrow(gg, _):
        plsc.store_scatter(idxc, [jnp.full((16,), gg, jnp.int32), 90 + lane],
                           zeros16, mask=lane < 6)
        return 0

    lax.fori_loop(0, ng, padrow, 0)

    def rchunk(s, _):
        pltpu.sync_copy(idx_hbm.at[wid, pl.ds(s * ichunk, ichunk), :], ibuf)

        def rerank(t2, _):
            p = s * ichunk + t2
            i16 = ibuf[t2]
            nbx = plsc.load_gather(posx_v, [i16])
            nby = plsc.load_gather(posy_v, [i16])
            qi = jnp.full((16,), r0 - b_base, jnp.int32) + p
            qxv = plsc.load_gather(posx_v, [qi])
            qyv = plsc.load_gather(posy_v, [qi])
            ddx = qxv - nbx
            ddy = qyv - nby
            d = ddx * ddx + ddy * ddy
            d = jnp.where(lane < _KSEL, d, inf)
            _, si = plsc.sort_key_val(d, i16)
            plsc.store_scatter(
                idxc,
                [jnp.full((16,), p // _SG, jnp.int32),
                 (p % _SG) * K + lane],
                si + b_base, mask=lane < K)
            return 0

        lax.fori_loop(0, ichunk, rerank, 0)
        return 0

    lax.fori_loop(0, span // ichunk, rchunk, 0)

    in_sems = (sem_in0, sem_in1)
    out_sems = (sem_out0, sem_out1)

    def issue_gathers(g, buf):
        pltpu.async_copy(x_hbm.at[idxc.at[g]], ring.at[buf], in_sems[buf])

    def drain_gathers(g, buf):
        pltpu.make_async_copy(
            x_hbm.at[idxc.at[g]], ring.at[buf], in_sems[buf]).wait()

    lane_c = jnp.minimum(lane, K - 1)

    def compute_group(g, buf):
        def point(t, _):
            i16 = plsc.load_gather(
                idxc, [jnp.full((16,), g, jnp.int32), t * K + lane_c]
            ) - b_base
            nbx = plsc.load_gather(posx_v, [i16])
            nby = plsc.load_gather(posy_v, [i16])
            qi = jnp.full((16,), r0 - b_base, jnp.int32) + (g * _SG + t)
            qxv = plsc.load_gather(posx_v, [qi])
            qyv = plsc.load_gather(posy_v, [qi])
            relx = qxv - nbx
            rely = qyv - nby
            rqx = jnp.clip(relx.astype(jnp.int32) + REL_POS_WIDTH,
                           0, TABLE_WIDTH - 1)
            rqy = jnp.clip(rely.astype(jnp.int32) + REL_POS_WIDTH,
                           0, TABLE_WIDTH - 1)
            pe = rqy * TABLE_WIDTH + rqx
            for i in range(INNER):
                wbuf[pl.ds(8 + i * 16, 16)] = plsc.load_gather(
                    wt_v, [pe + i * _T2PAD])
            wbc = [[plsc.load_gather(
                        wbuf, [jnp.full((16,), 8 + i * 16 + kk, jnp.int32)])
                    for kk in range(K)] for i in range(INNER)]
            for c in range(dim // 16):
                accs = [None] * INNER
                for kk in range(K):
                    xk = ring[buf, t * K + kk, pl.ds(c * 16, 16)]
                    for i in range(INNER):
                        contrib = wbc[i][kk] * xk
                        accs[i] = contrib if accs[i] is None \
                            else accs[i] + contrib
                for i in range(INNER):
                    fbuf[buf, t, pl.ds(i * dim + c * 16, 16)] = accs[i]
            return 0
        lax.fori_loop(0, _SG, point, 0)

    def out_copy(g, buf):
        return pltpu.make_async_copy(
            fbuf.at[buf],
            out_hbm.at[pl.ds(r0 + g * _SG, _SG), :],
            out_sems[buf],
        )

    issue_gathers(0, 0)

    def process(g, buf, issue_next, wait_out):
        if issue_next:
            issue_gathers(g + 1, 1 - buf)
        drain_gathers(g, buf)
        if wait_out:
            @pl.when(g >= 2)
            def _():
                out_copy(g - 2, buf).wait()
        compute_group(g, buf)
        out_copy(g, buf).start()

    def pipe(i, _):
        process(2 * i, 0, True, True)
        process(2 * i + 1, 1, True, True)
        return 0

    lax.fori_loop(0, (ng - 1) // 2, pipe, 0)
    process(ng - 1, 0, False, True)
    out_copy(ng - 2, 1).wait()
    out_copy(ng - 1, 0).wait()


def _sc_feat(x_flat, posx, posy, idx_r, wt_flat):
    n_total, dim = x_flat.shape
    span = n_total // _NW
    mesh = plsc.VectorSubcoreMesh(core_axis_name="c", subcore_axis_name="s")
    f = pl.kernel(
        _sc_feat_body,
        out_type=jax.ShapeDtypeStruct((n_total, INNER * dim), jnp.float32),
        mesh=mesh,
        compiler_params=pltpu.CompilerParams(use_tc_tiling_on_sc=False,
                                             needs_layout_passes=False),
        scratch_types=[
            pltpu.VMEM((posx.shape[1],), jnp.float32),
            pltpu.VMEM((posx.shape[1],), jnp.float32),
            pltpu.VMEM((min(125, span), 16), jnp.int32),
            pltpu.VMEM((span // _SG, 96), jnp.int32),
            pltpu.VMEM((wt_flat.shape[0],), jnp.float32),
            pltpu.VMEM((80,), jnp.float32),
            pltpu.VMEM((2, 96, dim), jnp.float32),
            pltpu.VMEM((2, _SG, INNER * dim), jnp.float32),
            pltpu.SemaphoreType.DMA,
            pltpu.SemaphoreType.DMA,
            pltpu.SemaphoreType.DMA,
            pltpu.SemaphoreType.DMA,
        ],
    )
    return f(x_flat, posx, posy, idx_r, wt_flat)


def _ln_matmul_kernel(f_ref, g2_ref, be2_ref, w_ref, bl_ref, o_ref):
    f = f_ref[...]
    m = jnp.mean(f, axis=-1, keepdims=True)
    d = f - m
    v = jnp.mean(d * d, axis=-1, keepdims=True)
    y = d * lax.rsqrt(v + 1e-5) * g2_ref[...] + be2_ref[...]
    o_ref[...] = (
        jnp.dot(y, w_ref[...], preferred_element_type=jnp.float32) + bl_ref[...]
    )


def _ln_matmul(feat, g2, be2, W, bL):
    M, C_in = feat.shape
    C_out = W.shape[1]
    R = next(r for r in (800, 512, 256, 128, 64, 32, 16, 8) if M % r == 0)
    grid = (M // R,)
    return pl.pallas_call(
        _ln_matmul_kernel,
        grid=grid,
        in_specs=[
            pl.BlockSpec((R, C_in), lambda i: (i, 0)),
            pl.BlockSpec((C_in,), lambda i: (0,)),
            pl.BlockSpec((C_in,), lambda i: (0,)),
            pl.BlockSpec((C_in, C_out), lambda i: (0, 0)),
            pl.BlockSpec((C_out,), lambda i: (0,)),
        ],
        out_specs=pl.BlockSpec((R, C_out), lambda i: (i, 0)),
        out_shape=jax.ShapeDtypeStruct((M, C_out), jnp.float32),
    )(feat, g2, be2, W, bL)


def kernel(x, pos, pre_table, w1, b1, g1, be1, g2, be2, W, bL):
    b, n, c = x.shape
    idx16 = _knn12(pos)
    wt = _weights_table(pre_table, w1, b1, g1, be1)
    wt_flat = wt[:INNER].reshape(-1)
    idx_r = idx16.reshape(_NW, (b * n) // _NW, 16)
    feat = _sc_feat(
        x.reshape(b * n, c),
        pos[..., 0], pos[..., 1],
        idx_r, wt_flat,
    )
    out = _ln_matmul(feat, g2, be2, W, bL)
    return out.reshape(b, n, W.shape[1])
